# Initial kernel scaffold; baseline (speedup 1.0000x reference)
import jax
import jax.numpy as jnp
from jax.experimental import pallas as pl


def kernel(positions, hidden_states, ln1_w, wqkv, wo, ln2_w, gate_w, w_gate, w_up, w_down):
    raise NotImplementedError("write your pallas kernel here")


# speedup vs baseline: 2.4680x; 2.4680x over previous
import functools

import jax
import jax.numpy as jnp
from jax import lax
from jax.experimental import pallas as pl
from jax.experimental.pallas import tpu as pltpu
from jax.experimental.pallas import tpu_sc as plsc

T = 2048
HIDDEN = 1024
NUM_HEADS = 16
NUM_KV_HEADS = 4
HEAD_DIM = 64
NUM_EXPERTS = 64
MOE_INTER = 512
ROPE_THETA = 10000.0
EPS = 1e-6

BLK = 32
P = 4096
NB = P // BLK

_HIGH = lax.Precision.DEFAULT


def _rmsnorm_body(x_ref, w_ref, o_ref):
    x = x_ref[...]
    v = jnp.mean(x * x, axis=-1, keepdims=True)
    o_ref[...] = x * lax.rsqrt(v + EPS) * w_ref[...]


def _rmsnorm(x, w2d):
    return pl.pallas_call(
        _rmsnorm_body,
        out_shape=jax.ShapeDtypeStruct(x.shape, x.dtype),
    )(x, w2d)


def _attn_body(h1_ref, wq_ref, wk_ref, wv_ref, cos_ref, sin_ref, o_ref):
    h1 = h1_ref[...]
    cos = cos_ref[...]
    sin = sin_ref[...]

    def rope(x):
        x1 = x[:, : HEAD_DIM // 2]
        x2 = x[:, HEAD_DIM // 2:]
        return jnp.concatenate([x1 * cos - x2 * sin, x2 * cos + x1 * sin], axis=1)

    dn = (((1,), (1,)), ((), ()))
    q = rope(lax.dot_general(h1, wq_ref[0], dn, precision=_HIGH))
    k = rope(lax.dot_general(h1, wk_ref[0], dn, precision=_HIGH))
    v = lax.dot_general(h1, wv_ref[0], dn, precision=_HIGH)

    s = lax.dot_general(q, k, dn, precision=_HIGH) * (HEAD_DIM ** -0.5)
    ri = lax.broadcasted_iota(jnp.int32, (T, T), 0)
    ci = lax.broadcasted_iota(jnp.int32, (T, T), 1)
    s = jnp.where(ci <= ri, s, -1e9)
    m = jnp.max(s, axis=-1, keepdims=True)
    p = jnp.exp(s - m)
    p = p / jnp.sum(p, axis=-1, keepdims=True)
    o_ref[0] = lax.dot_general(p, v, (((1,), (0,)), ((), ())), precision=_HIGH)


def _attention(h1, wq3, wk3, wv3, cos, sin):
    return pl.pallas_call(
        _attn_body,
        grid=(NUM_HEADS,),
        in_specs=[
            pl.BlockSpec((T, HIDDEN), lambda h: (0, 0)),
            pl.BlockSpec((1, HEAD_DIM, HIDDEN), lambda h: (h, 0, 0)),
            pl.BlockSpec((1, HEAD_DIM, HIDDEN), lambda h: (h // (NUM_HEADS // NUM_KV_HEADS), 0, 0)),
            pl.BlockSpec((1, HEAD_DIM, HIDDEN), lambda h: (h // (NUM_HEADS // NUM_KV_HEADS), 0, 0)),
            pl.BlockSpec((T, HEAD_DIM // 2), lambda h: (0, 0)),
            pl.BlockSpec((T, HEAD_DIM // 2), lambda h: (0, 0)),
        ],
        out_specs=pl.BlockSpec((1, T, HEAD_DIM), lambda h: (h, 0, 0)),
        out_shape=jax.ShapeDtypeStruct((NUM_HEADS, T, HEAD_DIM), jnp.float32),
    )(h1, wq3, wk3, wv3, cos, sin)


def _post_body(o_ref, wo_ref, x0_ref, w2_ref, gw_ref, r2_ref, h2_ref, eid_ref):
    dn = (((1,), (1,)), ((), ()))
    attn = x0_ref[...]
    for h in range(NUM_HEADS):
        attn = attn + lax.dot_general(
            o_ref[h], wo_ref[h], (((1,), (0,)), ((), ())), precision=_HIGH)
    r2 = attn
    r2_ref[...] = r2
    v = jnp.mean(r2 * r2, axis=-1, keepdims=True)
    h2 = r2 * lax.rsqrt(v + EPS) * w2_ref[...]
    h2_ref[...] = h2
    logits = lax.dot_general(h2, gw_ref[...], dn, precision=_HIGH)
    m = jnp.max(logits, axis=-1, keepdims=True)
    ci = lax.broadcasted_iota(jnp.int32, logits.shape, 1)
    cand = jnp.where(logits == m, ci, NUM_EXPERTS)
    eid_ref[...] = jnp.min(cand, axis=-1, keepdims=True)


_POST_ROWS = 256


def _post_attn(o3, woh, x0, w2d, gate_w):
    return pl.pallas_call(
        _post_body,
        grid=(T // _POST_ROWS,),
        in_specs=[
            pl.BlockSpec((NUM_HEADS, _POST_ROWS, HEAD_DIM), lambda i: (0, i, 0)),
            pl.BlockSpec((NUM_HEADS, HEAD_DIM, HIDDEN), lambda i: (0, 0, 0)),
            pl.BlockSpec((_POST_ROWS, HIDDEN), lambda i: (i, 0)),
            pl.BlockSpec((1, HIDDEN), lambda i: (0, 0)),
            pl.BlockSpec((NUM_EXPERTS, HIDDEN), lambda i: (0, 0)),
        ],
        out_specs=(
            pl.BlockSpec((_POST_ROWS, HIDDEN), lambda i: (i, 0)),
            pl.BlockSpec((_POST_ROWS, HIDDEN), lambda i: (i, 0)),
            pl.BlockSpec((_POST_ROWS, 1), lambda i: (i, 0)),
        ),
        out_shape=(
            jax.ShapeDtypeStruct((T, HIDDEN), jnp.float32),
            jax.ShapeDtypeStruct((T, HIDDEN), jnp.float32),
            jax.ShapeDtypeStruct((T, 1), jnp.int32),
        ),
    )(o3, woh, x0, w2d, gate_w)


def _moe_body(blk_ref, xs_ref, wg_ref, wu_ref, wd_ref, o_ref):
    del blk_ref
    dn = (((1,), (1,)), ((), ()))
    x = xs_ref[...]
    a = lax.dot_general(x, wg_ref[0], dn, precision=_HIGH)
    b = lax.dot_general(x, wu_ref[0], dn, precision=_HIGH)
    h = a * (1.0 / (1.0 + jnp.exp(-a))) * b
    o_ref[...] = lax.dot_general(h, wd_ref[0], dn, precision=_HIGH)


def _moe_ffn(blk_eid, xs, w_gate, w_up, w_down):
    grid_spec = pltpu.PrefetchScalarGridSpec(
        num_scalar_prefetch=1,
        grid=(NB,),
        in_specs=[
            pl.BlockSpec((BLK, HIDDEN), lambda b, blk: (b, 0)),
            pl.BlockSpec((1, MOE_INTER, HIDDEN), lambda b, blk: (blk[b], 0, 0)),
            pl.BlockSpec((1, MOE_INTER, HIDDEN), lambda b, blk: (blk[b], 0, 0)),
            pl.BlockSpec((1, HIDDEN, MOE_INTER), lambda b, blk: (blk[b], 0, 0)),
        ],
        out_specs=pl.BlockSpec((BLK, HIDDEN), lambda b, blk: (b, 0)),
    )
    return pl.pallas_call(
        _moe_body,
        grid_spec=grid_spec,
        out_shape=jax.ShapeDtypeStruct((P, HIDDEN), jnp.float32),
    )(blk_eid, xs, w_gate, w_up, w_down)


def _sc_gather(table, idx):
    n = idx.shape[0]
    d = table.shape[1]
    info = plsc.get_sparse_core_info()
    nw = info.num_cores * info.num_subcores
    b_per_w = n // nw
    ch = min(32, b_per_w)
    mesh = plsc.VectorSubcoreMesh(core_axis_name="c", subcore_axis_name="s")

    @functools.partial(
        pl.kernel,
        mesh=mesh,
        out_type=jax.ShapeDtypeStruct((n, d), table.dtype),
        scratch_types=[
            pltpu.VMEM((b_per_w,), jnp.int32),
            pltpu.VMEM((ch, d), table.dtype),
            pltpu.SemaphoreType.DMA,
        ],
    )
    def gk(table_hbm, idx_hbm, out_hbm, idx_v, rows_v, sem):
        wid = lax.axis_index("s") * info.num_cores + lax.axis_index("c")
        base = wid * b_per_w
        pltpu.sync_copy(idx_hbm.at[pl.ds(base, b_per_w)], idx_v)
        for j in range(b_per_w // ch):
            pltpu.async_copy(table_hbm.at[idx_v.at[pl.ds(j * ch, ch)]], rows_v, sem).wait()
            pltpu.sync_copy(rows_v, out_hbm.at[pl.ds(base + j * ch, ch)])

    return gk(table, idx)


def _add_body(a_ref, b_ref, o_ref):
    o_ref[...] = a_ref[...] + b_ref[...]


def _residual_add(a, b):
    return pl.pallas_call(
        _add_body,
        out_shape=jax.ShapeDtypeStruct(a.shape, a.dtype),
    )(a, b)


def kernel(positions, hidden_states, ln1_w, wqkv, wo, ln2_w, gate_w, w_gate, w_up, w_down):
    x = hidden_states
    inv_freq = 1.0 / (ROPE_THETA ** (jnp.arange(0, HEAD_DIM, 2, dtype=jnp.float32) / HEAD_DIM))
    freqs = positions.astype(jnp.float32)[:, None] * inv_freq[None, :]
    cos = jnp.cos(freqs)
    sin = jnp.sin(freqs)

    q_size = NUM_HEADS * HEAD_DIM
    kv_size = NUM_KV_HEADS * HEAD_DIM
    wq3 = wqkv[:q_size].reshape(NUM_HEADS, HEAD_DIM, HIDDEN)
    wk3 = wqkv[q_size:q_size + kv_size].reshape(NUM_KV_HEADS, HEAD_DIM, HIDDEN)
    wv3 = wqkv[q_size + kv_size:].reshape(NUM_KV_HEADS, HEAD_DIM, HIDDEN)

    woh = wo.reshape(HIDDEN, NUM_HEADS, HEAD_DIM).transpose(1, 2, 0)

    h1 = _rmsnorm(x, ln1_w.reshape(1, HIDDEN))
    o3 = _attention(h1, wq3, wk3, wv3, cos, sin)
    r2, h2, eid2 = _post_attn(o3, woh, x, ln2_w.reshape(1, HIDDEN), gate_w)

    eid = eid2[:, 0]
    order = jnp.argsort(eid).astype(jnp.int32)
    sorted_eid = eid[order]
    counts = jnp.bincount(eid, length=NUM_EXPERTS).astype(jnp.int32)
    blocks = (counts + BLK - 1) // BLK
    pad_start = (jnp.cumsum(blocks) - blocks).astype(jnp.int32) * BLK
    seg_start = (jnp.cumsum(counts) - counts).astype(jnp.int32)
    ranks = jnp.arange(T, dtype=jnp.int32) - seg_start[sorted_eid]
    dest = pad_start[sorted_eid] + ranks
    src = jnp.full((P,), T, jnp.int32).at[dest].set(order)
    blk_eid = jnp.zeros((NB,), jnp.int32).at[dest // BLK].set(sorted_eid)
    gidx = jnp.zeros((T,), jnp.int32).at[order].set(dest)

    h2ext = jnp.concatenate([h2, jnp.zeros((1, HIDDEN), jnp.float32)], axis=0)
    xs = _sc_gather(h2ext, src)
    ys = _moe_ffn(blk_eid, xs, w_gate, w_up, w_down)
    moe_out = _sc_gather(ys, gidx)
    return _residual_add(r2, moe_out)
